# baseline (speedup 1.0000x reference)
import jax
import jax.numpy as jnp
from jax import lax
from jax.experimental import pallas as pl
from jax.experimental.pallas import tpu as pltpu
from jax.experimental.pallas import tpu_sc as plsc

VOCAB = 100000
D = 128
B = 16384
NCTX = 4
NNEG = 5

NC = 2
NS = 16
NW = NC * NS
BPW = B // NW
K = 64
NCHUNK = BPW // K
NSET = 3


def _sc_body(cen, ctx, neg, in_emb, out_emb, pos_hbm, neg_hbm,
             cen_b, ctxr_b, negr_b, cidx_b, nidx_b, u_b, vs_b, ns_b,
             pos_sb, neg_sb, raw_sems, base_sems, add_sems, out_sems):
    cid = lax.axis_index("c")
    sid = lax.axis_index("s")
    wid = sid * NC + cid
    base0 = wid * BPW
    iota16 = jnp.arange(16, dtype=jnp.int32)
    zf = jnp.zeros((16,), jnp.float32)

    raw_cps = {}
    base_cps = {}
    add_cps = {}
    out_cps = {}

    def fire_raw(t):
        s = t % NSET
        b = base0 + t * K
        raw_cps[t] = [
            pltpu.async_copy(cen.at[pl.ds(b, K)], cen_b[s], raw_sems[s]),
            pltpu.async_copy(ctx.at[pl.ds(b * NCTX, K * NCTX)], ctxr_b[s],
                             raw_sems[s]),
            pltpu.async_copy(neg.at[pl.ds(b * NNEG, K * NNEG)], negr_b[s],
                             raw_sems[s]),
        ]

    def extract(t):
        s = t % NSET
        for cp in raw_cps.pop(t):
            cp.wait()
        for g in range(K // 16):
            for c in range(NCTX):
                flat = (iota16 + (g * 16)) * NCTX + c
                cidx_b[s][c][pl.ds(g * 16, 16)] = plsc.load_gather(
                    ctxr_b[s], [flat])
            for c in range(NNEG):
                flat = (iota16 + (g * 16)) * NNEG + c
                nidx_b[s][c][pl.ds(g * 16, 16)] = plsc.load_gather(
                    negr_b[s], [flat])

    def fire_base(t):
        s = t % NSET
        base_cps[t] = [
            pltpu.async_copy(in_emb.at[cen_b[s]], u_b[s], base_sems[s]),
            pltpu.async_copy(out_emb.at[cidx_b[s][0]], vs_b[s], base_sems[s]),
            pltpu.async_copy(out_emb.at[nidx_b[s][0]], ns_b[s], base_sems[s]),
        ]

    def fire_adds(t):
        s = t % NSET
        for cp in base_cps.pop(t):
            cp.wait()
        cps = []
        for c in range(1, NCTX):
            cps.append(pltpu.async_copy(out_emb.at[cidx_b[s][c]], vs_b[s],
                                        add_sems[s], add=True))
        for c in range(1, NNEG):
            cps.append(pltpu.async_copy(out_emb.at[nidx_b[s][c]], ns_b[s],
                                        add_sems[s], add=True))
        add_cps[t] = cps

    def compute(t):
        s = t % NSET
        o = t % 2
        for cp in add_cps.pop(t):
            cp.wait()
        if t - 2 in out_cps:
            for cp in out_cps.pop(t - 2):
                cp.wait()
        u_v, vs_v, ns_v = u_b[s], vs_b[s], ns_b[s]
        rows = [iota16 + (g * 16) for g in range(K // 16)]

        def dstep(i, carry):
            accs, col = carry
            c0 = col
            c1 = col + 1
            out = []
            for g in range(K // 16):
                ap, an = accs[g]
                for cc in (c0, c1):
                    uu = plsc.load_gather(u_v, [rows[g], cc])
                    vv = plsc.load_gather(vs_v, [rows[g], cc])
                    nn = plsc.load_gather(ns_v, [rows[g], cc])
                    ap = ap + uu * vv
                    an = an + uu * nn
                out.append((ap, an))
            return (tuple(out), col + 2)

        accs0 = tuple((zf, zf) for _ in range(K // 16))
        col0 = jnp.zeros((16,), jnp.int32)
        accs, _ = lax.fori_loop(0, D // 2, dstep, (accs0, col0))
        for g in range(K // 16):
            ap, an = accs[g]
            pos_sb[o][pl.ds(g * 16, 16)] = ap * (1.0 / NCTX)
            neg_sb[o][pl.ds(g * 16, 16)] = an * (1.0 / NNEG)
        b = base0 + t * K
        out_cps[t] = [
            pltpu.async_copy(pos_sb[o], pos_hbm.at[pl.ds(b, K)], out_sems[o]),
            pltpu.async_copy(neg_sb[o], neg_hbm.at[pl.ds(b, K)], out_sems[o]),
        ]

    fire_raw(0)
    fire_raw(1)
    fire_raw(2)
    extract(0)
    fire_base(0)
    extract(1)
    fire_base(1)
    fire_adds(0)
    for t in range(NCHUNK):
        if t + 2 < NCHUNK:
            extract(t + 2)
            fire_base(t + 2)
        if t + 3 < NCHUNK:
            fire_raw(t + 3)
        if t + 1 < NCHUNK:
            fire_adds(t + 1)
        compute(t)
    for t in sorted(out_cps):
        for cp in out_cps[t]:
            cp.wait()


def _scores_sc(cen, ctx, neg, in_emb, out_emb):
    mesh = plsc.VectorSubcoreMesh(core_axis_name="c", subcore_axis_name="s",
                                  num_cores=NC, num_subcores=NS)
    f32 = jnp.float32
    i32 = jnp.int32
    run = pl.kernel(
        _sc_body,
        out_type=(jax.ShapeDtypeStruct((B,), f32),
                  jax.ShapeDtypeStruct((B,), f32)),
        mesh=mesh,
        compiler_params=pltpu.CompilerParams(needs_layout_passes=False),
        scratch_types=[
            [pltpu.VMEM((K,), i32) for _ in range(NSET)],
            [pltpu.VMEM((K * NCTX,), i32) for _ in range(NSET)],
            [pltpu.VMEM((K * NNEG,), i32) for _ in range(NSET)],
            [[pltpu.VMEM((K,), i32) for _ in range(NCTX)]
             for _ in range(NSET)],
            [[pltpu.VMEM((K,), i32) for _ in range(NNEG)]
             for _ in range(NSET)],
            [pltpu.VMEM((K, D), f32) for _ in range(NSET)],
            [pltpu.VMEM((K, D), f32) for _ in range(NSET)],
            [pltpu.VMEM((K, D), f32) for _ in range(NSET)],
            [pltpu.VMEM((K,), f32) for _ in range(2)],
            [pltpu.VMEM((K,), f32) for _ in range(2)],
            [pltpu.SemaphoreType.DMA for _ in range(NSET)],
            [pltpu.SemaphoreType.DMA for _ in range(NSET)],
            [pltpu.SemaphoreType.DMA for _ in range(NSET)],
            [pltpu.SemaphoreType.DMA for _ in range(2)],
        ],
    )
    return run(cen, ctx, neg, in_emb, out_emb)


def _loss_body(pos_ref, neg_ref, o_ref):
    pos = pos_ref[...]
    neg = neg_ref[...]
    loss = jax.nn.log_sigmoid(pos) + jax.nn.log_sigmoid(-neg)
    o_ref[0, 0] = -jnp.mean(loss)


def _loss_tc(pos, neg):
    out = pl.pallas_call(
        _loss_body,
        out_shape=jax.ShapeDtypeStruct((1, 1), jnp.float32),
        in_specs=[pl.BlockSpec(memory_space=pltpu.VMEM),
                  pl.BlockSpec(memory_space=pltpu.VMEM)],
        out_specs=pl.BlockSpec(memory_space=pltpu.SMEM),
    )(pos.reshape(128, 128), neg.reshape(128, 128))
    return out[0, 0]


@jax.jit
def kernel(centers, context, neg_context, in_emb, out_emb):
    centers = centers.astype(jnp.int32)
    context = context.astype(jnp.int32).reshape(B * NCTX)
    neg_context = neg_context.astype(jnp.int32).reshape(B * NNEG)
    pos, neg = _scores_sc(centers, context, neg_context, in_emb, out_emb)
    return _loss_tc(pos, neg)


# speedup vs baseline: 1.6814x; 1.6814x over previous
import jax
import jax.numpy as jnp
from jax import lax
from jax.experimental import pallas as pl
from jax.experimental.pallas import tpu as pltpu
from jax.experimental.pallas import tpu_sc as plsc

VOCAB = 100000
D = 128
B = 16384
NCTX = 4
NNEG = 5

NC = 2
NS = 16
NW = NC * NS
BPW = B // NW
K = 64
NCHUNK = BPW // K
NSET = 3


def _sc_body(cen, ctx, neg, in_emb, out_emb, pos_hbm, neg_hbm,
             cen_b, ctxr_b, negr_b, cidx_b, nidx_b, u_b, vs_b, ns_b,
             pos_sb, neg_sb, raw_sems, base_sems, add_sems, out_sems):
    cid = lax.axis_index("c")
    sid = lax.axis_index("s")
    wid = sid * NC + cid
    base0 = wid * BPW
    iota16 = jnp.arange(16, dtype=jnp.int32)
    zf = jnp.zeros((16,), jnp.float32)

    raw_cps = {}
    base_cps = {}
    add_cps = {}
    out_cps = {}

    def fire_raw(t):
        s = t % NSET
        b = base0 + t * K
        raw_cps[t] = [
            pltpu.async_copy(cen.at[pl.ds(b, K)], cen_b[s], raw_sems[s]),
            pltpu.async_copy(ctx.at[pl.ds(b * NCTX, K * NCTX)], ctxr_b[s],
                             raw_sems[s]),
            pltpu.async_copy(neg.at[pl.ds(b * NNEG, K * NNEG)], negr_b[s],
                             raw_sems[s]),
        ]

    def extract(t):
        s = t % NSET
        for cp in raw_cps.pop(t):
            cp.wait()
        for g in range(K // 16):
            for c in range(NCTX):
                flat = (iota16 + (g * 16)) * NCTX + c
                cidx_b[s][c][pl.ds(g * 16, 16)] = plsc.load_gather(
                    ctxr_b[s], [flat])
            for c in range(NNEG):
                flat = (iota16 + (g * 16)) * NNEG + c
                nidx_b[s][c][pl.ds(g * 16, 16)] = plsc.load_gather(
                    negr_b[s], [flat])

    def fire_base(t):
        s = t % NSET
        base_cps[t] = [
            pltpu.async_copy(in_emb.at[cen_b[s]], u_b[s], base_sems[s]),
            pltpu.async_copy(out_emb.at[cidx_b[s][0]], vs_b[s], base_sems[s]),
            pltpu.async_copy(out_emb.at[nidx_b[s][0]], ns_b[s], base_sems[s]),
        ]

    def fire_adds(t):
        s = t % NSET
        for cp in base_cps.pop(t):
            cp.wait()
        cps = []
        for c in range(1, NCTX):
            cps.append(pltpu.async_copy(out_emb.at[cidx_b[s][c]], vs_b[s],
                                        add_sems[s], add=True))
        for c in range(1, NNEG):
            cps.append(pltpu.async_copy(out_emb.at[nidx_b[s][c]], ns_b[s],
                                        add_sems[s], add=True))
        add_cps[t] = cps

    def compute(t):
        s = t % NSET
        o = t % 2
        for cp in add_cps.pop(t):
            cp.wait()
        if t - 2 in out_cps:
            for cp in out_cps.pop(t - 2):
                cp.wait()
        u_v, vs_v, ns_v = u_b[s], vs_b[s], ns_b[s]

        def elem(k, carry):
            accp = u_v[k, pl.ds(0, 16)] * vs_v[k, pl.ds(0, 16)]
            accn = u_v[k, pl.ds(0, 16)] * ns_v[k, pl.ds(0, 16)]
            for j in range(1, D // 16):
                uu = u_v[k, pl.ds(16 * j, 16)]
                accp = accp + uu * vs_v[k, pl.ds(16 * j, 16)]
                accn = accn + uu * ns_v[k, pl.ds(16 * j, 16)]
            pos_sb[o][k, pl.ds(0, 16)] = accp
            neg_sb[o][k, pl.ds(0, 16)] = accn
            return carry

        lax.fori_loop(0, K, elem, 0)
        b = base0 + t * K
        out_cps[t] = [
            pltpu.async_copy(pos_sb[o], pos_hbm.at[pl.ds(b, K)], out_sems[o]),
            pltpu.async_copy(neg_sb[o], neg_hbm.at[pl.ds(b, K)], out_sems[o]),
        ]

    fire_raw(0)
    fire_raw(1)
    fire_raw(2)
    extract(0)
    fire_base(0)
    extract(1)
    fire_base(1)
    fire_adds(0)
    for t in range(NCHUNK):
        if t + 2 < NCHUNK:
            extract(t + 2)
            fire_base(t + 2)
        if t + 3 < NCHUNK:
            fire_raw(t + 3)
        if t + 1 < NCHUNK:
            fire_adds(t + 1)
        compute(t)
    for t in sorted(out_cps):
        for cp in out_cps[t]:
            cp.wait()


def _scores_sc(cen, ctx, neg, in_emb, out_emb):
    mesh = plsc.VectorSubcoreMesh(core_axis_name="c", subcore_axis_name="s",
                                  num_cores=NC, num_subcores=NS)
    f32 = jnp.float32
    i32 = jnp.int32
    run = pl.kernel(
        _sc_body,
        out_type=(jax.ShapeDtypeStruct((B, 16), f32),
                  jax.ShapeDtypeStruct((B, 16), f32)),
        mesh=mesh,
        compiler_params=pltpu.CompilerParams(needs_layout_passes=False),
        scratch_types=[
            [pltpu.VMEM((K,), i32) for _ in range(NSET)],
            [pltpu.VMEM((K * NCTX,), i32) for _ in range(NSET)],
            [pltpu.VMEM((K * NNEG,), i32) for _ in range(NSET)],
            [[pltpu.VMEM((K,), i32) for _ in range(NCTX)]
             for _ in range(NSET)],
            [[pltpu.VMEM((K,), i32) for _ in range(NNEG)]
             for _ in range(NSET)],
            [pltpu.VMEM((K, D), f32) for _ in range(NSET)],
            [pltpu.VMEM((K, D), f32) for _ in range(NSET)],
            [pltpu.VMEM((K, D), f32) for _ in range(NSET)],
            [pltpu.VMEM((K, 16), f32) for _ in range(2)],
            [pltpu.VMEM((K, 16), f32) for _ in range(2)],
            [pltpu.SemaphoreType.DMA for _ in range(NSET)],
            [pltpu.SemaphoreType.DMA for _ in range(NSET)],
            [pltpu.SemaphoreType.DMA for _ in range(NSET)],
            [pltpu.SemaphoreType.DMA for _ in range(2)],
        ],
    )
    return run(cen, ctx, neg, in_emb, out_emb)


def _loss_body(pos_ref, neg_ref, o_ref):
    pos = jnp.sum(pos_ref[...], axis=1) * (1.0 / NCTX)
    neg = jnp.sum(neg_ref[...], axis=1) * (1.0 / NNEG)
    loss = jax.nn.log_sigmoid(pos) + jax.nn.log_sigmoid(-neg)
    o_ref[0, 0] = -jnp.mean(loss)


def _loss_tc(pos_part, neg_part):
    out = pl.pallas_call(
        _loss_body,
        out_shape=jax.ShapeDtypeStruct((1, 1), jnp.float32),
        in_specs=[pl.BlockSpec(memory_space=pltpu.VMEM),
                  pl.BlockSpec(memory_space=pltpu.VMEM)],
        out_specs=pl.BlockSpec(memory_space=pltpu.SMEM),
    )(pos_part, neg_part)
    return out[0, 0]


@jax.jit
def kernel(centers, context, neg_context, in_emb, out_emb):
    centers = centers.astype(jnp.int32)
    context = context.astype(jnp.int32).reshape(B * NCTX)
    neg_context = neg_context.astype(jnp.int32).reshape(B * NNEG)
    pos, neg = _scores_sc(centers, context, neg_context, in_emb, out_emb)
    return _loss_tc(pos, neg)


# speedup vs baseline: 2.5460x; 1.5142x over previous
import jax
import jax.numpy as jnp
from jax import lax
from jax.experimental import pallas as pl
from jax.experimental.pallas import tpu as pltpu
from jax.experimental.pallas import tpu_sc as plsc

VOCAB = 100000
D = 128
B = 16384
NCTX = 4
NNEG = 5
NIDX = 1 + NCTX + NNEG

NC = 2
NS = 16
NW = NC * NS
BPW = B // NW
K = 64
NCHUNK = BPW // K
NSET = 3


def _sc_body(idx_hbm, in_emb, out_emb, pos_hbm, neg_hbm,
             idx_b, u_b, vs_b, ns_b, pos_sb, neg_sb,
             raw_sems, base_sems, add_sems, out_sems):
    cid = lax.axis_index("c")
    sid = lax.axis_index("s")
    wid = sid * NC + cid
    base0 = wid * BPW
    iota16 = jnp.arange(16, dtype=jnp.int32)

    raw_cps = {}
    base_cps = {}
    add_cps = {}
    out_cps = {}

    def fire_raw(t):
        s = t % NSET
        b = base0 + t * K
        raw_cps[t] = [
            pltpu.async_copy(idx_hbm.at[r, pl.ds(b, K)], idx_b[s][r],
                             raw_sems[s])
            for r in range(NIDX)
        ]

    def fire_base(t):
        s = t % NSET
        for cp in raw_cps.pop(t):
            cp.wait()
        base_cps[t] = [
            pltpu.async_copy(in_emb.at[idx_b[s][0]], u_b[s], base_sems[s]),
            pltpu.async_copy(out_emb.at[idx_b[s][1]], vs_b[s], base_sems[s]),
            pltpu.async_copy(out_emb.at[idx_b[s][1 + NCTX]], ns_b[s],
                             base_sems[s]),
        ]

    def fire_adds(t):
        s = t % NSET
        for cp in base_cps.pop(t):
            cp.wait()
        cps = []
        for r in range(2, 1 + NCTX):
            cps.append(pltpu.async_copy(out_emb.at[idx_b[s][r]], vs_b[s],
                                        add_sems[s], add=True))
        for r in range(2 + NCTX, NIDX):
            cps.append(pltpu.async_copy(out_emb.at[idx_b[s][r]], ns_b[s],
                                        add_sems[s], add=True))
        add_cps[t] = cps

    def compute(t):
        s = t % NSET
        o = t % 2
        for cp in add_cps.pop(t):
            cp.wait()
        if t - 2 in out_cps:
            for cp in out_cps.pop(t - 2):
                cp.wait()
        u_v, vs_v, ns_v = u_b[s], vs_b[s], ns_b[s]

        def grp(g, carry):
            sp_v = jnp.zeros((16,), jnp.float32)
            sn_v = jnp.zeros((16,), jnp.float32)
            for i in range(16):
                k = g * 16 + i
                accp = u_v[k, pl.ds(0, 16)] * vs_v[k, pl.ds(0, 16)]
                accn = u_v[k, pl.ds(0, 16)] * ns_v[k, pl.ds(0, 16)]
                for j in range(1, D // 16):
                    uu = u_v[k, pl.ds(16 * j, 16)]
                    accp = accp + uu * vs_v[k, pl.ds(16 * j, 16)]
                    accn = accn + uu * ns_v[k, pl.ds(16 * j, 16)]
                sp = jnp.sum(accp) * (1.0 / NCTX)
                sn = jnp.sum(accn) * (1.0 / NNEG)
                sp_v = jnp.where(iota16 == i, sp, sp_v)
                sn_v = jnp.where(iota16 == i, sn, sn_v)
            pos_sb[o][pl.ds(g * 16, 16)] = sp_v
            neg_sb[o][pl.ds(g * 16, 16)] = sn_v
            return carry

        lax.fori_loop(0, K // 16, grp, 0)
        b = base0 + t * K
        out_cps[t] = [
            pltpu.async_copy(pos_sb[o], pos_hbm.at[pl.ds(b, K)], out_sems[o]),
            pltpu.async_copy(neg_sb[o], neg_hbm.at[pl.ds(b, K)], out_sems[o]),
        ]

    fire_raw(0)
    fire_raw(1)
    fire_raw(2)
    fire_base(0)
    fire_base(1)
    fire_adds(0)
    for t in range(NCHUNK):
        if t + 2 < NCHUNK:
            fire_base(t + 2)
        if t + 3 < NCHUNK:
            fire_raw(t + 3)
        if t + 1 < NCHUNK:
            fire_adds(t + 1)
        compute(t)
    for t in sorted(out_cps):
        for cp in out_cps[t]:
            cp.wait()


def _scores_sc(idx_all, in_emb, out_emb):
    mesh = plsc.VectorSubcoreMesh(core_axis_name="c", subcore_axis_name="s",
                                  num_cores=NC, num_subcores=NS)
    f32 = jnp.float32
    i32 = jnp.int32
    run = pl.kernel(
        _sc_body,
        out_type=(jax.ShapeDtypeStruct((B,), f32),
                  jax.ShapeDtypeStruct((B,), f32)),
        mesh=mesh,
        compiler_params=pltpu.CompilerParams(needs_layout_passes=False),
        scratch_types=[
            [[pltpu.VMEM((K,), i32) for _ in range(NIDX)]
             for _ in range(NSET)],
            [pltpu.VMEM((K, D), f32) for _ in range(NSET)],
            [pltpu.VMEM((K, D), f32) for _ in range(NSET)],
            [pltpu.VMEM((K, D), f32) for _ in range(NSET)],
            [pltpu.VMEM((K,), f32) for _ in range(2)],
            [pltpu.VMEM((K,), f32) for _ in range(2)],
            [pltpu.SemaphoreType.DMA for _ in range(NSET)],
            [pltpu.SemaphoreType.DMA for _ in range(NSET)],
            [pltpu.SemaphoreType.DMA for _ in range(NSET)],
            [pltpu.SemaphoreType.DMA for _ in range(2)],
        ],
    )
    return run(idx_all, in_emb, out_emb)


def _loss_body(pos_ref, neg_ref, o_ref):
    pos = pos_ref[...]
    neg = neg_ref[...]
    loss = jax.nn.log_sigmoid(pos) + jax.nn.log_sigmoid(-neg)
    o_ref[0, 0] = -jnp.mean(loss)


def _loss_tc(pos, neg):
    out = pl.pallas_call(
        _loss_body,
        out_shape=jax.ShapeDtypeStruct((1, 1), jnp.float32),
        in_specs=[pl.BlockSpec(memory_space=pltpu.VMEM),
                  pl.BlockSpec(memory_space=pltpu.VMEM)],
        out_specs=pl.BlockSpec(memory_space=pltpu.SMEM),
    )(pos.reshape(128, 128), neg.reshape(128, 128))
    return out[0, 0]


@jax.jit
def kernel(centers, context, neg_context, in_emb, out_emb):
    centers = centers.astype(jnp.int32)
    context = context.astype(jnp.int32)
    neg_context = neg_context.astype(jnp.int32)
    idx_all = jnp.concatenate(
        [centers[None, :], context.T, neg_context.T], axis=0)
    pos, neg = _scores_sc(idx_all, in_emb, out_emb)
    return _loss_tc(pos, neg)


# speedup vs baseline: 2.6348x; 1.0349x over previous
import jax
import jax.numpy as jnp
from jax import lax
from jax.experimental import pallas as pl
from jax.experimental.pallas import tpu as pltpu
from jax.experimental.pallas import tpu_sc as plsc

VOCAB = 100000
D = 128
B = 16384
NCTX = 4
NNEG = 5
NIDX = 1 + NCTX + NNEG

NC = 2
NS = 16
NW = NC * NS
BPW = B // NW
K = 128
NCHUNK = BPW // K
NSET = 2


def _sc_body(idx_hbm, in_emb, out_emb, pos_hbm, neg_hbm,
             idx_b, u_b, vs_b, ns_b, pos_sb, neg_sb,
             raw_sems, g_sems, out_sems):
    cid = lax.axis_index("c")
    sid = lax.axis_index("s")
    wid = sid * NC + cid
    base0 = wid * BPW
    iota16 = jnp.arange(16, dtype=jnp.int32)
    zf = jnp.zeros((16,), jnp.float32)

    raw_cps = {}
    g_cps = {}
    out_cps = {}

    def fire_raw(t):
        s = t % NSET
        b = base0 + t * K
        raw_cps[t] = [
            pltpu.async_copy(idx_hbm.at[r, pl.ds(b, K)], idx_b[s][r],
                             raw_sems[s])
            for r in range(NIDX)
        ]

    def zero_set(s):
        def z(k, carry):
            for j in range(D // 16):
                vs_b[s][k, pl.ds(16 * j, 16)] = zf
                ns_b[s][k, pl.ds(16 * j, 16)] = zf
            return carry
        lax.fori_loop(0, K, z, 0)

    def fire_gathers(t):
        s = t % NSET
        for cp in raw_cps.pop(t):
            cp.wait()
        cps = [pltpu.async_copy(in_emb.at[idx_b[s][0]], u_b[s], g_sems[s])]
        for r in range(1, 1 + NCTX):
            cps.append(pltpu.async_copy(out_emb.at[idx_b[s][r]], vs_b[s],
                                        g_sems[s], add=True))
        for r in range(1 + NCTX, NIDX):
            cps.append(pltpu.async_copy(out_emb.at[idx_b[s][r]], ns_b[s],
                                        g_sems[s], add=True))
        g_cps[t] = cps

    def compute(t):
        s = t % NSET
        o = t % 2
        for cp in g_cps.pop(t):
            cp.wait()
        if t - 2 in out_cps:
            for cp in out_cps.pop(t - 2):
                cp.wait()
        u_v, vs_v, ns_v = u_b[s], vs_b[s], ns_b[s]

        def grp(g, carry):
            sp_v = jnp.zeros((16,), jnp.float32)
            sn_v = jnp.zeros((16,), jnp.float32)
            for i in range(16):
                k = g * 16 + i
                accp = u_v[k, pl.ds(0, 16)] * vs_v[k, pl.ds(0, 16)]
                accn = u_v[k, pl.ds(0, 16)] * ns_v[k, pl.ds(0, 16)]
                vs_v[k, pl.ds(0, 16)] = zf
                ns_v[k, pl.ds(0, 16)] = zf
                for j in range(1, D // 16):
                    uu = u_v[k, pl.ds(16 * j, 16)]
                    accp = accp + uu * vs_v[k, pl.ds(16 * j, 16)]
                    accn = accn + uu * ns_v[k, pl.ds(16 * j, 16)]
                    vs_v[k, pl.ds(16 * j, 16)] = zf
                    ns_v[k, pl.ds(16 * j, 16)] = zf
                sp = jnp.sum(accp) * (1.0 / NCTX)
                sn = jnp.sum(accn) * (1.0 / NNEG)
                sp_v = jnp.where(iota16 == i, sp, sp_v)
                sn_v = jnp.where(iota16 == i, sn, sn_v)
            pos_sb[o][pl.ds(g * 16, 16)] = sp_v
            neg_sb[o][pl.ds(g * 16, 16)] = sn_v
            return carry

        lax.fori_loop(0, K // 16, grp, 0)
        b = base0 + t * K
        out_cps[t] = [
            pltpu.async_copy(pos_sb[o], pos_hbm.at[pl.ds(b, K)], out_sems[o]),
            pltpu.async_copy(neg_sb[o], neg_hbm.at[pl.ds(b, K)], out_sems[o]),
        ]

    fire_raw(0)
    fire_raw(1)
    zero_set(0)
    fire_gathers(0)
    zero_set(1)
    for t in range(NCHUNK):
        if t + 1 < NCHUNK:
            fire_gathers(t + 1)
        compute(t)
        if t + 2 < NCHUNK:
            fire_raw(t + 2)
    for t in sorted(out_cps):
        for cp in out_cps[t]:
            cp.wait()


def _scores_sc(idx_all, in_emb, out_emb):
    mesh = plsc.VectorSubcoreMesh(core_axis_name="c", subcore_axis_name="s",
                                  num_cores=NC, num_subcores=NS)
    f32 = jnp.float32
    i32 = jnp.int32
    run = pl.kernel(
        _sc_body,
        out_type=(jax.ShapeDtypeStruct((B,), f32),
                  jax.ShapeDtypeStruct((B,), f32)),
        mesh=mesh,
        compiler_params=pltpu.CompilerParams(needs_layout_passes=False),
        scratch_types=[
            [[pltpu.VMEM((K,), i32) for _ in range(NIDX)]
             for _ in range(NSET)],
            [pltpu.VMEM((K, D), f32) for _ in range(NSET)],
            [pltpu.VMEM((K, D), f32) for _ in range(NSET)],
            [pltpu.VMEM((K, D), f32) for _ in range(NSET)],
            [pltpu.VMEM((K,), f32) for _ in range(2)],
            [pltpu.VMEM((K,), f32) for _ in range(2)],
            [pltpu.SemaphoreType.DMA for _ in range(NSET)],
            [pltpu.SemaphoreType.DMA for _ in range(NSET)],
            [pltpu.SemaphoreType.DMA for _ in range(2)],
        ],
    )
    return run(idx_all, in_emb, out_emb)


def _loss_body(pos_ref, neg_ref, o_ref):
    pos = pos_ref[...]
    neg = neg_ref[...]
    loss = jax.nn.log_sigmoid(pos) + jax.nn.log_sigmoid(-neg)
    o_ref[0, 0] = -jnp.mean(loss)


def _loss_tc(pos, neg):
    out = pl.pallas_call(
        _loss_body,
        out_shape=jax.ShapeDtypeStruct((1, 1), jnp.float32),
        in_specs=[pl.BlockSpec(memory_space=pltpu.VMEM),
                  pl.BlockSpec(memory_space=pltpu.VMEM)],
        out_specs=pl.BlockSpec(memory_space=pltpu.SMEM),
    )(pos.reshape(128, 128), neg.reshape(128, 128))
    return out[0, 0]


@jax.jit
def kernel(centers, context, neg_context, in_emb, out_emb):
    centers = centers.astype(jnp.int32)
    context = context.astype(jnp.int32)
    neg_context = neg_context.astype(jnp.int32)
    idx_all = jnp.concatenate(
        [centers[None, :], context.T, neg_context.T], axis=0)
    pos, neg = _scores_sc(idx_all, in_emb, out_emb)
    return _loss_tc(pos, neg)
